# baseline (speedup 1.0000x reference)
import functools

import jax
import jax.numpy as jnp
from jax import lax
from jax.experimental import pallas as pl
from jax.experimental.pallas import tpu as pltpu
from jax.experimental.pallas import tpu_sc as plsc

SEQ = 128
D = 64


@jax.jit
def _sc_embed(x, emb, pos_t):
    b_total = x.shape[0]
    info = plsc.get_sparse_core_info()
    nc, ns = info.num_cores, info.num_subcores
    nw = nc * ns
    b_per_w = b_total // nw

    mesh = plsc.VectorSubcoreMesh(core_axis_name="c", subcore_axis_name="s")

    @functools.partial(
        pl.kernel,
        mesh=mesh,
        compiler_params=pltpu.CompilerParams(
            use_tc_tiling_on_sc=False, needs_layout_passes=False),
        out_type=jax.ShapeDtypeStruct((b_total, D, SEQ), jnp.float32),
        scratch_types=[
            pltpu.VMEM((b_per_w, SEQ), jnp.int32),
            pltpu.VMEM((D, SEQ), jnp.float32),
            pltpu.VMEM((SEQ, D), jnp.float32),
            pltpu.VMEM((SEQ, D), jnp.float32),
            pltpu.VMEM((D, SEQ), jnp.float32),
            pltpu.VMEM((D, SEQ), jnp.float32),
            pltpu.VMEM((32, 16, 16), jnp.float32),
            pltpu.SemaphoreType.DMA,
            pltpu.SemaphoreType.DMA,
        ],
    )
    def k(x_hbm, emb_hbm, pos_hbm, out_hbm, idx_v, pos_v,
          buf0, buf1, ob0, ob1, pos_diag, sem0, sem1):
        wid = lax.axis_index("s") * nc + lax.axis_index("c")
        base = wid * b_per_w

        pltpu.sync_copy(x_hbm.at[pl.ds(base, b_per_w)], idx_v)
        pltpu.sync_copy(pos_hbm, pos_v)

        def start_gather(c, buf, sem):
            pltpu.make_async_copy(emb_hbm.at[idx_v.at[c]], buf, sem).start()

        start_gather(0, buf0, sem0)
        start_gather(1, buf1, sem1)

        iota = lax.iota(jnp.int32, 16)
        diags = tuple((iota + j) & 15 for j in range(16))

        def pre_body(blk, acc):
            rows = iota + (blk & 7) * 16
            vb16 = (blk >> 3) * 16
            for j in range(16):
                pos_diag[blk, j] = plsc.load_gather(
                    pos_v, [diags[j] + vb16, rows])
            return acc
        lax.fori_loop(0, 32, pre_body, 0)

        def process(c, buf, ob, sem):
            pltpu.make_async_copy(emb_hbm.at[idx_v.at[c]], buf, sem).wait()

            def t_body(blk, acc):
                kk16 = (blk & 7) * 16
                vb16 = (blk >> 3) * 16
                rows = iota + kk16
                for j in range(16):
                    vr = diags[j] + vb16
                    g = plsc.load_gather(buf, [rows, vr])
                    plsc.store_scatter(ob, [vr, rows], g + pos_diag[blk, j])
                return acc
            lax.fori_loop(0, 32, t_body, 0, unroll=2)

            pltpu.sync_copy(ob, out_hbm.at[base + c])

        def body(i, carry):
            c = i * 2
            process(c, buf0, ob0, sem0)
            start_gather(lax.rem(c + 2, b_per_w), buf0, sem0)
            process(c + 1, buf1, ob1, sem1)
            start_gather(lax.rem(c + 3, b_per_w), buf1, sem1)
            return carry

        lax.fori_loop(0, b_per_w // 2, body, 0)

        pltpu.make_async_copy(emb_hbm.at[idx_v.at[0]], buf0, sem0).wait()
        pltpu.make_async_copy(emb_hbm.at[idx_v.at[1]], buf1, sem1).wait()

    return k(x, emb, pos_t)


def kernel(x, embedding, pos_embedding):
    out_t = _sc_embed(x.astype(jnp.int32), embedding, pos_embedding.T)
    return out_t.transpose(0, 2, 1)


# speedup vs baseline: 1.0085x; 1.0085x over previous
import functools

import jax
import jax.numpy as jnp
from jax import lax
from jax.experimental import pallas as pl
from jax.experimental.pallas import tpu as pltpu
from jax.experimental.pallas import tpu_sc as plsc

SEQ = 128
D = 64


@jax.jit
def _sc_embed(x, emb, pos_t):
    b_total = x.shape[0]
    info = plsc.get_sparse_core_info()
    nc, ns = info.num_cores, info.num_subcores
    nw = nc * ns
    b_per_w = b_total // nw

    mesh = plsc.VectorSubcoreMesh(core_axis_name="c", subcore_axis_name="s")

    @functools.partial(
        pl.kernel,
        mesh=mesh,
        compiler_params=pltpu.CompilerParams(
            use_tc_tiling_on_sc=False, needs_layout_passes=False),
        out_type=jax.ShapeDtypeStruct((b_total, D, SEQ), jnp.float32),
        scratch_types=[
            pltpu.VMEM((b_per_w, SEQ), jnp.int32),
            pltpu.VMEM((D, SEQ), jnp.float32),
            pltpu.VMEM((SEQ, D), jnp.float32),
            pltpu.VMEM((SEQ, D), jnp.float32),
            pltpu.VMEM((D, SEQ), jnp.float32),
            pltpu.VMEM((D, SEQ), jnp.float32),
            pltpu.VMEM((2, 4, 64, 16), jnp.float32),
            pltpu.SemaphoreType.DMA,
            pltpu.SemaphoreType.DMA,
        ],
    )
    def k(x_hbm, emb_hbm, pos_hbm, out_hbm, idx_v, pos_v,
          buf0, buf1, ob0, ob1, pos_diag, sem0, sem1):
        wid = lax.axis_index("s") * nc + lax.axis_index("c")
        base = wid * b_per_w

        pltpu.sync_copy(x_hbm.at[pl.ds(base, b_per_w)], idx_v)
        pltpu.sync_copy(pos_hbm, pos_v)

        def start_gather(c, buf, sem):
            pltpu.make_async_copy(emb_hbm.at[idx_v.at[c]], buf, sem).start()

        start_gather(0, buf0, sem0)
        start_gather(1, buf1, sem1)

        iota = lax.iota(jnp.int32, 16)
        vv = tuple((iota * 17 + 16 * s) & 63 for s in range(4))

        def pre_body(d, acc):
            for h in range(2):
                for s in range(4):
                    t = ((vv[s] + d) & 63) + 64 * h
                    pos_diag[h, s, d] = plsc.load_gather(pos_v, [vv[s], t])
            return acc
        lax.fori_loop(0, 64, pre_body, 0)

        def process(c, buf, ob, sem):
            pltpu.make_async_copy(emb_hbm.at[idx_v.at[c]], buf, sem).wait()

            def t_body(d, acc):
                for h in range(2):
                    for s in range(4):
                        t = ((vv[s] + d) & 63) + 64 * h
                        g = plsc.load_gather(buf, [t, vv[s]])
                        plsc.store_scatter(ob, [vv[s], t],
                                           g + pos_diag[h, s, d])
                return acc
            lax.fori_loop(0, 64, t_body, 0, unroll=2)

            pltpu.sync_copy(ob, out_hbm.at[base + c])

        def body(i, carry):
            c = i * 2
            process(c, buf0, ob0, sem0)
            start_gather(lax.rem(c + 2, b_per_w), buf0, sem0)
            process(c + 1, buf1, ob1, sem1)
            start_gather(lax.rem(c + 3, b_per_w), buf1, sem1)
            return carry

        lax.fori_loop(0, b_per_w // 2, body, 0)

        pltpu.make_async_copy(emb_hbm.at[idx_v.at[0]], buf0, sem0).wait()
        pltpu.make_async_copy(emb_hbm.at[idx_v.at[1]], buf1, sem1).wait()

    return k(x, emb, pos_t)


def kernel(x, embedding, pos_embedding):
    out_t = _sc_embed(x.astype(jnp.int32), embedding, pos_embedding.T)
    return out_t.transpose(0, 2, 1)


# speedup vs baseline: 1.3263x; 1.3152x over previous
import functools

import jax
import jax.numpy as jnp
from jax import lax
from jax.experimental import pallas as pl
from jax.experimental.pallas import tpu as pltpu
from jax.experimental.pallas import tpu_sc as plsc

SEQ = 128
D = 64


@jax.jit
def _sc_embed(x, emb, pos_t):
    b_total = x.shape[0]
    info = plsc.get_sparse_core_info()
    nc, ns = info.num_cores, info.num_subcores
    nw = nc * ns
    b_per_w = b_total // nw

    mesh = plsc.VectorSubcoreMesh(core_axis_name="c", subcore_axis_name="s")

    @functools.partial(
        pl.kernel,
        mesh=mesh,
        compiler_params=pltpu.CompilerParams(
            use_tc_tiling_on_sc=False, needs_layout_passes=False),
        out_type=jax.ShapeDtypeStruct((b_total, D, SEQ), jnp.float32),
        scratch_types=[
            pltpu.VMEM((b_per_w, SEQ), jnp.int32),
            pltpu.VMEM((D, SEQ), jnp.float32),
            pltpu.VMEM((SEQ, D), jnp.float32),
            pltpu.VMEM((SEQ, D), jnp.float32),
            pltpu.VMEM((D, SEQ), jnp.float32),
            pltpu.VMEM((D, SEQ), jnp.float32),
            pltpu.VMEM((2, 4, 64, 16), jnp.float32),
            pltpu.SemaphoreType.DMA,
            pltpu.SemaphoreType.DMA,
        ],
    )
    def k(x_hbm, emb_hbm, pos_hbm, out_hbm, idx_v, pos_v,
          buf0, buf1, ob0, ob1, pos_diag, sem0, sem1):
        wid = lax.axis_index("s") * nc + lax.axis_index("c")
        base = wid * b_per_w

        pltpu.sync_copy(x_hbm.at[pl.ds(base, b_per_w)], idx_v)
        pltpu.sync_copy(pos_hbm, pos_v)

        def start_gather(c, buf, sem):
            pltpu.make_async_copy(emb_hbm.at[idx_v.at[c]], buf, sem).start()

        start_gather(0, buf0, sem0)
        start_gather(1, buf1, sem1)

        iota = lax.iota(jnp.int32, 16)
        vv = tuple((iota * 17 + 16 * s) & 63 for s in range(4))

        def pre_body(d, acc):
            for h in range(2):
                for s in range(4):
                    t = ((vv[s] + d) & 63) + 64 * h
                    pos_diag[h, s, d] = plsc.load_gather(pos_v, [vv[s], t])
            return acc
        lax.fori_loop(0, 64, pre_body, 0)

        def process(c, buf, ob, sem):
            pltpu.make_async_copy(emb_hbm.at[idx_v.at[c]], buf, sem).wait()

            @plsc.parallel_loop(0, 64, unroll=2)
            def t_body(d):
                for h in range(2):
                    for s in range(4):
                        t = ((vv[s] + d) & 63) + 64 * h
                        g = plsc.load_gather(buf, [t, vv[s]])
                        plsc.store_scatter(ob, [vv[s], t],
                                           g + pos_diag[h, s, d])

            pltpu.sync_copy(ob, out_hbm.at[base + c])

        def body(i, carry):
            c = i * 2
            process(c, buf0, ob0, sem0)
            start_gather(lax.rem(c + 2, b_per_w), buf0, sem0)
            process(c + 1, buf1, ob1, sem1)
            start_gather(lax.rem(c + 3, b_per_w), buf1, sem1)
            return carry

        lax.fori_loop(0, b_per_w // 2, body, 0)

        pltpu.make_async_copy(emb_hbm.at[idx_v.at[0]], buf0, sem0).wait()
        pltpu.make_async_copy(emb_hbm.at[idx_v.at[1]], buf1, sem1).wait()

    return k(x, emb, pos_t)


def kernel(x, embedding, pos_embedding):
    out_t = _sc_embed(x.astype(jnp.int32), embedding, pos_embedding.T)
    return out_t.transpose(0, 2, 1)
